# Initial kernel scaffold; baseline (speedup 1.0000x reference)
import jax
import jax.numpy as jnp
from jax.experimental import pallas as pl


def kernel(x, gate_W, gate_b, cl_W1, cl_b1, cl_W2, cl_b2, exp_W1, exp_b1, exp_W2, exp_b2):
    raise NotImplementedError("write your pallas kernel here")


# speedup vs baseline: 1.2886x; 1.2886x over previous
import functools

import jax
import jax.numpy as jnp
import numpy as np
from jax.experimental import pallas as pl
from jax.experimental.pallas import tpu as pltpu

_TEMP = float(np.e)


def _routing_body(x_ref, gw_ref, gb_ref, c1w_ref, c1b_ref, c2w_ref, c2b_ref, w_ref):
    n, d = x_ref.shape
    e = gw_ref.shape[1]
    xb = x_ref[...].astype(jnp.bfloat16)
    scores = jnp.dot(xb, gw_ref[...].astype(jnp.bfloat16),
                     preferred_element_type=jnp.float32)
    scores = (scores + gb_ref[...][None, :]) / _TEMP
    smax = jnp.max(scores, axis=1, keepdims=True)
    ex = jnp.exp(scores - smax)
    probs = ex / jnp.sum(ex, axis=1, keepdims=True)
    c1 = jnp.dot(xb, c1w_ref[...].astype(jnp.bfloat16),
                 preferred_element_type=jnp.float32)
    c1 = jnp.maximum(c1 + c1b_ref[...][None, :], 0.0)
    c1b16 = c1.astype(jnp.bfloat16).astype(jnp.float32)
    w2 = c2w_ref[...].astype(jnp.bfloat16).astype(jnp.float32)
    pre = jnp.sum(c1b16 * w2[None, :, 0], axis=1, keepdims=True) + c2b_ref[0]
    clarity = 1.0 / (1.0 + jnp.exp(-pre))
    n_active = e - clarity * (e - 2)
    mean_act = jnp.sum(n_active) / n
    k = jnp.clip(jnp.floor(mean_act + 0.5).astype(jnp.int32), 2, e)
    rank = jnp.zeros((n, e), dtype=jnp.int32)
    lane = jax.lax.broadcasted_iota(jnp.int32, (n, e), 1)
    for j in range(e):
        pj = probs[:, j:j + 1]
        beats = (pj > probs) | ((pj == probs) & (j < lane))
        rank = rank + beats.astype(jnp.int32)
    mask = (rank < k).astype(jnp.float32)
    w = probs * mask
    w = w / (jnp.sum(w, axis=1, keepdims=True) + 1e-8)
    w_ref[...] = w


def _expert_body(x_ref, w1_ref, b1_ref, w2_ref, b2_ref, wts_ref, out_ref):
    eidx = pl.program_id(1)
    h = jnp.dot(x_ref[...], w1_ref[0], preferred_element_type=jnp.float32)
    h = jnp.maximum(h + b1_ref[0], 0.0)
    y = jnp.dot(h.astype(jnp.bfloat16), w2_ref[0],
                preferred_element_type=jnp.float32)
    y = y + b2_ref[0]
    ecols = jax.lax.broadcasted_iota(jnp.int32, (1, wts_ref.shape[1]), 1)
    sel = (ecols == eidx).astype(jnp.float32)
    wcol = jnp.sum(wts_ref[...] * sel, axis=1, keepdims=True)
    contrib = wcol * y

    @pl.when(eidx == 0)
    def _():
        out_ref[...] = contrib

    @pl.when(eidx != 0)
    def _():
        out_ref[...] = out_ref[...] + contrib


def kernel(x, gate_W, gate_b, cl_W1, cl_b1, cl_W2, cl_b2, exp_W1, exp_b1, exp_W2, exp_b2):
    n, d = x.shape
    e = gate_W.shape[1]
    h_dim = exp_W1.shape[2]
    o_dim = exp_W2.shape[2]

    weights = pl.pallas_call(
        _routing_body,
        out_shape=jax.ShapeDtypeStruct((n, e), jnp.float32),
    )(x, gate_W, gate_b, cl_W1, cl_b1, cl_W2, cl_b2)

    bn = min(n, 1024)
    ni = n // bn
    xb = x.astype(jnp.bfloat16)
    w1b = exp_W1.astype(jnp.bfloat16)
    w2b = exp_W2.astype(jnp.bfloat16)

    out = pl.pallas_call(
        _expert_body,
        grid=(ni, e),
        in_specs=[
            pl.BlockSpec((bn, d), lambda i, j: (i, 0)),
            pl.BlockSpec((1, d, h_dim), lambda i, j: (j, 0, 0)),
            pl.BlockSpec((1, 1, h_dim), lambda i, j: (j, 0, 0)),
            pl.BlockSpec((1, h_dim, o_dim), lambda i, j: (j, 0, 0)),
            pl.BlockSpec((1, 1, o_dim), lambda i, j: (j, 0, 0)),
            pl.BlockSpec((bn, e), lambda i, j: (i, 0)),
        ],
        out_specs=pl.BlockSpec((bn, o_dim), lambda i, j: (i, 0)),
        out_shape=jax.ShapeDtypeStruct((n, o_dim), jnp.float32),
        compiler_params=pltpu.CompilerParams(
            dimension_semantics=("arbitrary", "arbitrary"),
        ),
    )(xb, w1b, exp_b1.reshape(e, 1, h_dim), w2b, exp_b2.reshape(e, 1, o_dim), weights)
    return out


# speedup vs baseline: 1.3158x; 1.0211x over previous
import functools

import jax
import jax.numpy as jnp
import numpy as np
from jax.experimental import pallas as pl
from jax.experimental.pallas import tpu as pltpu

_TEMP = float(np.e)


def _routing_body(x_ref, gw_ref, gb_ref, c1w_ref, c1b_ref, c2w_ref, c2b_ref, w_ref):
    n, d = x_ref.shape
    e = gw_ref.shape[1]
    xb = x_ref[...].astype(jnp.bfloat16)
    scores = jnp.dot(xb, gw_ref[...].astype(jnp.bfloat16),
                     preferred_element_type=jnp.float32)
    scores = (scores + gb_ref[...][None, :]) / _TEMP
    smax = jnp.max(scores, axis=1, keepdims=True)
    ex = jnp.exp(scores - smax)
    probs = ex / jnp.sum(ex, axis=1, keepdims=True)
    c1 = jnp.dot(xb, c1w_ref[...].astype(jnp.bfloat16),
                 preferred_element_type=jnp.float32)
    c1 = jnp.maximum(c1 + c1b_ref[...][None, :], 0.0)
    c1b16 = c1.astype(jnp.bfloat16).astype(jnp.float32)
    w2 = c2w_ref[...].astype(jnp.bfloat16).astype(jnp.float32)
    pre = jnp.sum(c1b16 * w2[None, :, 0], axis=1, keepdims=True) + c2b_ref[0]
    clarity = 1.0 / (1.0 + jnp.exp(-pre))
    n_active = e - clarity * (e - 2)
    mean_act = jnp.sum(n_active) / n
    k = jnp.clip(jnp.floor(mean_act + 0.5).astype(jnp.int32), 2, e)
    rank = jnp.zeros((n, e), dtype=jnp.int32)
    lane = jax.lax.broadcasted_iota(jnp.int32, (n, e), 1)
    for j in range(e):
        pj = probs[:, j:j + 1]
        beats = (pj > probs) | ((pj == probs) & (j < lane))
        rank = rank + beats.astype(jnp.int32)
    mask = (rank < k).astype(jnp.float32)
    w = probs * mask
    w = w / (jnp.sum(w, axis=1, keepdims=True) + 1e-8)
    w_ref[...] = w


def _expert_body(x_ref, w1_ref, b1_ref, w2_ref, b2_ref, wts_ref, out_ref):
    eidx = pl.program_id(0)
    hc = pl.program_id(1)
    w1 = w1_ref[0].astype(jnp.bfloat16)
    h = jnp.dot(x_ref[...], w1, preferred_element_type=jnp.float32)
    h = jnp.maximum(h + b1_ref[0], 0.0)
    w2 = w2_ref[0].astype(jnp.bfloat16)
    y = jnp.dot(h.astype(jnp.bfloat16), w2, preferred_element_type=jnp.float32)
    y = jnp.where(hc == 0, y + b2_ref[0], y)
    ecols = jax.lax.broadcasted_iota(jnp.int32, (1, wts_ref.shape[1]), 1)
    sel = (ecols == eidx).astype(jnp.float32)
    wcol = jnp.sum(wts_ref[...] * sel, axis=1, keepdims=True)
    contrib = wcol * y

    @pl.when((eidx == 0) & (hc == 0))
    def _():
        out_ref[...] = contrib

    @pl.when((eidx != 0) | (hc != 0))
    def _():
        out_ref[...] = out_ref[...] + contrib


def kernel(x, gate_W, gate_b, cl_W1, cl_b1, cl_W2, cl_b2, exp_W1, exp_b1, exp_W2, exp_b2):
    n, d = x.shape
    e = gate_W.shape[1]
    h_dim = exp_W1.shape[2]
    o_dim = exp_W2.shape[2]

    weights = pl.pallas_call(
        _routing_body,
        out_shape=jax.ShapeDtypeStruct((n, e), jnp.float32),
    )(x, gate_W, gate_b, cl_W1, cl_b1, cl_W2, cl_b2)

    hcb = min(h_dim, 512)
    nhc = h_dim // hcb
    xb = x.astype(jnp.bfloat16)

    out = pl.pallas_call(
        _expert_body,
        grid=(e, nhc),
        in_specs=[
            pl.BlockSpec((n, d), lambda j, c: (0, 0)),
            pl.BlockSpec((1, d, hcb), lambda j, c: (j, 0, c)),
            pl.BlockSpec((1, 1, hcb), lambda j, c: (j, 0, c)),
            pl.BlockSpec((1, hcb, o_dim), lambda j, c: (j, c, 0)),
            pl.BlockSpec((1, 1, o_dim), lambda j, c: (j, 0, 0)),
            pl.BlockSpec((n, e), lambda j, c: (0, 0)),
        ],
        out_specs=pl.BlockSpec((n, o_dim), lambda j, c: (0, 0)),
        out_shape=jax.ShapeDtypeStruct((n, o_dim), jnp.float32),
        compiler_params=pltpu.CompilerParams(
            dimension_semantics=("arbitrary", "arbitrary"),
        ),
    )(xb, exp_W1, exp_b1.reshape(e, 1, h_dim), exp_W2, exp_b2.reshape(e, 1, o_dim), weights)
    return out
